# baseline (speedup 1.0000x reference)
import functools

import jax
import jax.numpy as jnp
from jax import lax
from jax.experimental import pallas as pl
from jax.experimental.pallas import tpu as pltpu
from jax.experimental.pallas import tpu_sc as plsc

B, S, D = 4, 2048, 1024
T = B * S
LANES = 16


def _build_kernel():
    info = plsc.get_sparse_core_info()
    nw = info.num_cores * info.num_subcores
    t_per_w = T // nw
    chunk = 32
    n_chunks = t_per_w // chunk

    @functools.partial(
        pl.kernel,
        mesh=plsc.VectorSubcoreMesh(core_axis_name="c", subcore_axis_name="s"),
        out_type=jax.ShapeDtypeStruct((T, D), jnp.float32),
        scratch_types=[
            pltpu.VMEM((chunk,), jnp.int32),
            pltpu.VMEM((chunk, D), jnp.float32),
            pltpu.VMEM((chunk, D), jnp.float32),
            pltpu.SemaphoreType.DMA,
        ],
    )
    def emb_kernel(ids_hbm, tok_hbm, pos_hbm, out_hbm, idx_v, tok_buf, pos_buf, sem):
        wid = lax.axis_index("s") * info.num_cores + lax.axis_index("c")
        base = wid * t_per_w
        pos_base = base % S

        def chunk_body(ci, carry):
            off = base + ci * chunk
            poff = pos_base + ci * chunk
            pltpu.sync_copy(ids_hbm.at[pl.ds(off, chunk)], idx_v)
            pltpu.sync_copy(pos_hbm.at[pl.ds(poff, chunk), :], pos_buf)
            pltpu.async_copy(tok_hbm.at[idx_v], tok_buf, sem).wait()

            def row_body(r, c2):
                for j in range(D // LANES):
                    sl = pl.ds(j * LANES, LANES)
                    plsc.addupdate(tok_buf.at[r, sl], pos_buf[r, sl])
                return c2

            lax.fori_loop(0, chunk, row_body, 0)
            pltpu.sync_copy(tok_buf, out_hbm.at[pl.ds(off, chunk), :])
            return carry

        lax.fori_loop(0, n_chunks, chunk_body, 0)

    return emb_kernel


_EMB_KERNEL = None


def kernel(input_ids, token_table, pos_table):
    global _EMB_KERNEL
    if _EMB_KERNEL is None:
        _EMB_KERNEL = _build_kernel()
    ids_flat = input_ids.reshape(T).astype(jnp.int32)
    out = _EMB_KERNEL(ids_flat, token_table, pos_table)
    return out.reshape(B, S, D)


# speedup vs baseline: 1.3866x; 1.3866x over previous
import functools

import jax
import jax.numpy as jnp
from jax import lax
from jax.experimental import pallas as pl
from jax.experimental.pallas import tpu as pltpu
from jax.experimental.pallas import tpu_sc as plsc

B, S, D = 4, 2048, 1024
T = B * S
LANES = 16
CH = 16


def _build_kernel():
    info = plsc.get_sparse_core_info()
    nc, ns = info.num_cores, info.num_subcores
    nw = nc * ns
    p_per_w = S // nw
    n_q = p_per_w // CH

    @functools.partial(
        pl.kernel,
        mesh=plsc.VectorSubcoreMesh(core_axis_name="c", subcore_axis_name="s"),
        out_type=jax.ShapeDtypeStruct((T, D), jnp.float32),
        scratch_types=[
            pltpu.VMEM((B, p_per_w), jnp.int32),
            pltpu.VMEM((p_per_w, D), jnp.float32),
            pltpu.VMEM((CH, D), jnp.float32),
            pltpu.VMEM((CH, D), jnp.float32),
            pltpu.VMEM((CH, D), jnp.float32),
            pltpu.SemaphoreType.DMA,
            pltpu.SemaphoreType.DMA,
            pltpu.SemaphoreType.DMA,
            pltpu.SemaphoreType.DMA,
            pltpu.SemaphoreType.DMA,
            pltpu.SemaphoreType.DMA,
            pltpu.SemaphoreType.DMA,
        ],
    )
    def emb_kernel(ids_hbm, tok_hbm, pos_hbm, out_hbm,
                   idx_v, pos_full, buf0, buf1, buf2,
                   psem, g0, g1, g2, s0, s1, s2):
        wid = lax.axis_index("s") * nc + lax.axis_index("c")
        p0 = wid * p_per_w

        bufs = (buf0, buf1, buf2)
        gsems = (g0, g1, g2)
        ssems = (s0, s1, s2)

        pos_cp = pltpu.make_async_copy(pos_hbm.at[pl.ds(p0, p_per_w), :],
                                       pos_full, psem)
        pos_cp.start()
        for b in range(B):
            pltpu.sync_copy(ids_hbm.at[pl.ds(b * S + p0, p_per_w)],
                            idx_v.at[b])

        work = [(b, q) for b in range(B) for q in range(n_q)]
        n_work = len(work)
        pend_gather = [None, None, None]
        pend_store = [None, None, None]

        def issue_gather(i):
            j = i % 3
            if pend_store[j] is not None:
                pend_store[j].wait()
                pend_store[j] = None
            b, q = work[i]
            h = pltpu.make_async_copy(
                tok_hbm.at[idx_v.at[b, pl.ds(q * CH, CH)]], bufs[j], gsems[j])
            h.start()
            pend_gather[j] = h

        issue_gather(0)
        issue_gather(1)
        pos_cp.wait()

        for i in range(n_work):
            j = i % 3
            b, q = work[i]
            pend_gather[j].wait()
            pend_gather[j] = None

            buf = bufs[j]

            def row_body(r, carry, _buf=buf, _q=q):
                for col in range(D // LANES):
                    sl = pl.ds(col * LANES, LANES)
                    plsc.addupdate(_buf.at[r, sl], pos_full[_q * CH + r, sl])
                return carry

            lax.fori_loop(0, CH, row_body, 0)

            h = pltpu.make_async_copy(
                buf, out_hbm.at[pl.ds(b * S + p0 + q * CH, CH), :], ssems[j])
            h.start()
            pend_store[j] = h

            if i + 2 < n_work:
                issue_gather(i + 2)

        for j in range(3):
            if pend_store[j] is not None:
                pend_store[j].wait()

    return emb_kernel


_EMB_KERNEL = None


def kernel(input_ids, token_table, pos_table):
    global _EMB_KERNEL
    if _EMB_KERNEL is None:
        _EMB_KERNEL = _build_kernel()
    ids_flat = input_ids.reshape(T).astype(jnp.int32)
    out = _EMB_KERNEL(ids_flat, token_table, pos_table)
    return out.reshape(B, S, D)


# speedup vs baseline: 1.5106x; 1.0894x over previous
import functools

import jax
import jax.numpy as jnp
from jax import lax
from jax.experimental import pallas as pl
from jax.experimental.pallas import tpu as pltpu
from jax.experimental.pallas import tpu_sc as plsc

B, S, D = 4, 2048, 1024
T = B * S
LANES = 16
CH = 8


def _build_kernel():
    info = plsc.get_sparse_core_info()
    nc, ns = info.num_cores, info.num_subcores
    nw = nc * ns
    p_per_w = S // nw
    n_q = p_per_w // CH

    @functools.partial(
        pl.kernel,
        mesh=plsc.VectorSubcoreMesh(core_axis_name="c", subcore_axis_name="s"),
        out_type=jax.ShapeDtypeStruct((T, D), jnp.float32),
        scratch_types=(
            [pltpu.VMEM((B, p_per_w), jnp.int32)]
            + [pltpu.VMEM((CH, D), jnp.float32) for _ in range(2 * B)]
            + [pltpu.VMEM((CH, D), jnp.float32) for _ in range(2)]
            + [pltpu.SemaphoreType.DMA for _ in range(6)]
        ),
    )
    def emb_kernel(ids_hbm, tok_hbm, pos_hbm, out_hbm,
                   idx_v, t00, t01, t02, t03, t10, t11, t12, t13,
                   pb0, pb1, g0, g1, p0s, p1s, s0, s1):
        wid = lax.axis_index("s") * nc + lax.axis_index("c")
        p0 = wid * p_per_w

        tbufs = ((t00, t01, t02, t03), (t10, t11, t12, t13))
        pbufs = (pb0, pb1)
        gsems = (g0, g1)
        psems = (p0s, p1s)
        ssems = (s0, s1)

        for b in range(B):
            pltpu.sync_copy(ids_hbm.at[pl.ds(b * S + p0, p_per_w)],
                            idx_v.at[b])

        pend_gather = [None, None]
        pend_store = [None, None]

        def issue_group(q):
            g = q % 2
            if pend_store[g] is not None:
                for h in pend_store[g]:
                    h.wait()
                pend_store[g] = None
            hs = []
            h = pltpu.make_async_copy(
                pos_hbm.at[pl.ds(p0 + q * CH, CH), :], pbufs[g], psems[g])
            h.start()
            hs.append(h)
            for b in range(B):
                h = pltpu.make_async_copy(
                    tok_hbm.at[idx_v.at[b, pl.ds(q * CH, CH)]],
                    tbufs[g][b], gsems[g])
                h.start()
                hs.append(h)
            pend_gather[g] = hs

        issue_group(0)
        issue_group(1)

        for q in range(n_q):
            g = q % 2
            for h in pend_gather[g]:
                h.wait()
            pend_gather[g] = None

            pb = pbufs[g]
            tb = tbufs[g]

            def row_body(r, carry, _pb=pb, _tb=tb):
                for half in range(2):
                    cols = range(half * 32, half * 32 + 32)
                    vs = [_pb[r, pl.ds(c * LANES, LANES)] for c in cols]
                    for b in range(B):
                        for v, c in zip(vs, cols):
                            plsc.addupdate(
                                _tb[b].at[r, pl.ds(c * LANES, LANES)], v)
                return carry

            lax.fori_loop(0, CH, row_body, 0)

            hs = []
            for b in range(B):
                h = pltpu.make_async_copy(
                    tb[b], out_hbm.at[pl.ds(b * S + p0 + q * CH, CH), :],
                    ssems[g])
                h.start()
                hs.append(h)
            pend_store[g] = hs

            if q + 2 < n_q:
                issue_group(q + 2)

        for g in range(2):
            if pend_store[g] is not None:
                for h in pend_store[g]:
                    h.wait()

    return emb_kernel


_EMB_KERNEL = None


def kernel(input_ids, token_table, pos_table):
    global _EMB_KERNEL
    if _EMB_KERNEL is None:
        _EMB_KERNEL = _build_kernel()
    ids_flat = input_ids.reshape(T).astype(jnp.int32)
    out = _EMB_KERNEL(ids_flat, token_table, pos_table)
    return out.reshape(B, S, D)


# speedup vs baseline: 1.5545x; 1.0291x over previous
import functools

import jax
import jax.numpy as jnp
from jax import lax
from jax.experimental import pallas as pl
from jax.experimental.pallas import tpu as pltpu
from jax.experimental.pallas import tpu_sc as plsc

B, S, D = 4, 2048, 1024
T = B * S
LANES = 16
CH = 8


def _build_kernel():
    info = plsc.get_sparse_core_info()
    nc, ns = info.num_cores, info.num_subcores
    nw = nc * ns
    p_per_w = S // nw
    n_q = p_per_w // CH

    @functools.partial(
        pl.kernel,
        mesh=plsc.VectorSubcoreMesh(core_axis_name="c", subcore_axis_name="s"),
        out_type=jax.ShapeDtypeStruct((T, D), jnp.float32),
        scratch_types=(
            [pltpu.VMEM((B, p_per_w), jnp.int32)]
            + [pltpu.VMEM((CH, D), jnp.float32) for _ in range(3 * B)]
            + [pltpu.VMEM((CH, D), jnp.float32) for _ in range(3)]
            + [pltpu.SemaphoreType.DMA for _ in range(9)]
        ),
    )
    def emb_kernel(ids_hbm, tok_hbm, pos_hbm, out_hbm,
                   idx_v, t00, t01, t02, t03, t10, t11, t12, t13,
                   t20, t21, t22, t23, pb0, pb1, pb2,
                   g0, g1, g2, p0s, p1s, p2s, s0, s1, s2):
        wid = lax.axis_index("s") * nc + lax.axis_index("c")
        p0 = wid * p_per_w

        tbufs = ((t00, t01, t02, t03), (t10, t11, t12, t13),
                 (t20, t21, t22, t23))
        pbufs = (pb0, pb1, pb2)
        gsems = (g0, g1, g2)
        psems = (p0s, p1s, p2s)
        ssems = (s0, s1, s2)

        for b in range(B):
            pltpu.sync_copy(ids_hbm.at[pl.ds(b * S + p0, p_per_w)],
                            idx_v.at[b])

        pend_gather = [None, None, None]
        pend_store = [None, None, None]

        def issue_group(q):
            g = q % 3
            if pend_store[g] is not None:
                for h in pend_store[g]:
                    h.wait()
                pend_store[g] = None
            hs = []
            h = pltpu.make_async_copy(
                pos_hbm.at[pl.ds(p0 + q * CH, CH), :], pbufs[g], psems[g])
            h.start()
            hs.append(h)
            for b in range(B):
                h = pltpu.make_async_copy(
                    tok_hbm.at[idx_v.at[b, pl.ds(q * CH, CH)]],
                    tbufs[g][b], gsems[g])
                h.start()
                hs.append(h)
            pend_gather[g] = hs

        issue_group(0)
        issue_group(1)

        for q in range(n_q):
            g = q % 3
            for h in pend_gather[g]:
                h.wait()
            pend_gather[g] = None

            if q + 2 < n_q:
                issue_group(q + 2)

            pb = pbufs[g]
            tb = tbufs[g]

            def row_body(r, carry, _pb=pb, _tb=tb):
                for half in range(2):
                    cols = range(half * 32, half * 32 + 32)
                    vs = [_pb[r, pl.ds(c * LANES, LANES)] for c in cols]
                    for b in range(B):
                        for v, c in zip(vs, cols):
                            plsc.addupdate(
                                _tb[b].at[r, pl.ds(c * LANES, LANES)], v)
                return carry

            lax.fori_loop(0, CH, row_body, 0)

            hs = []
            for b in range(B):
                h = pltpu.make_async_copy(
                    tb[b], out_hbm.at[pl.ds(b * S + p0 + q * CH, CH), :],
                    ssems[g])
                h.start()
                hs.append(h)
            pend_store[g] = hs

        for g in range(3):
            if pend_store[g] is not None:
                for h in pend_store[g]:
                    h.wait()

    return emb_kernel


_EMB_KERNEL = None


def kernel(input_ids, token_table, pos_table):
    global _EMB_KERNEL
    if _EMB_KERNEL is None:
        _EMB_KERNEL = _build_kernel()
    ids_flat = input_ids.reshape(T).astype(jnp.int32)
    out = _EMB_KERNEL(ids_flat, token_table, pos_table)
    return out.reshape(B, S, D)


# speedup vs baseline: 1.5763x; 1.0140x over previous
import functools

import jax
import jax.numpy as jnp
from jax import lax
from jax.experimental import pallas as pl
from jax.experimental.pallas import tpu as pltpu
from jax.experimental.pallas import tpu_sc as plsc

B, S, D = 4, 2048, 1024
T = B * S
CH = 8
NGRP = 3
LANES = 16


def _build_kernel():
    info = plsc.get_sparse_core_info()
    nc, ns = info.num_cores, info.num_subcores
    nw = nc * ns
    p_per_w = S // nw
    n_q = p_per_w // CH
    rows_g = B * CH

    @functools.partial(
        pl.kernel,
        mesh=plsc.VectorSubcoreMesh(core_axis_name="c", subcore_axis_name="s"),
        out_type=jax.ShapeDtypeStruct((T, D), jnp.float32),
        scratch_types=(
            [pltpu.VMEM((n_q, rows_g), jnp.int32)]
            + [pltpu.VMEM((rows_g, D), jnp.float32) for _ in range(NGRP)]
            + [pltpu.VMEM((CH, D), jnp.float32) for _ in range(NGRP)]
            + [pltpu.SemaphoreType.DMA for _ in range(3 * NGRP)]
        ),
    )
    def emb_kernel(ids_hbm, tok_hbm, pos_hbm, out_hbm,
                   idx_v, tb0, tb1, tb2, pb0, pb1, pb2,
                   g0, g1, g2, p0s, p1s, p2s, s0, s1, s2):
        cid = lax.axis_index("c")
        sid = lax.axis_index("s")
        wid = sid * nc + cid
        p0 = wid * p_per_w

        tbufs = (tb0, tb1, tb2)
        pbufs = (pb0, pb1, pb2)
        gsems = (g0, g1, g2)
        psems = (p0s, p1s, p2s)
        ssems = (s0, s1, s2)

        pltpu.sync_copy(
            ids_hbm.at[pl.ds(pl.multiple_of(wid * n_q, 8), n_q), :], idx_v)

        pend_in = [None] * NGRP
        pend_store = [None] * NGRP

        def issue_chunk(q):
            g = q % NGRP
            if pend_store[g] is not None:
                for h in pend_store[g]:
                    h.wait()
                pend_store[g] = None
            hp = pltpu.make_async_copy(
                pos_hbm.at[pl.ds(pl.multiple_of(p0 + q * CH, 8), CH), :],
                pbufs[g], psems[g])
            hp.start()
            hg = pltpu.make_async_copy(
                tok_hbm.at[idx_v.at[q]], tbufs[g], gsems[g])
            hg.start()
            pend_in[g] = (hp, hg)

        for q in range(NGRP):
            issue_chunk(q)

        for q in range(n_q):
            g = q % NGRP
            for h in pend_in[g]:
                h.wait()
            pend_in[g] = None

            pb = pbufs[g]
            tb = tbufs[g]

            def row_body(r, carry, _pb=pb, _tb=tb):
                for c in range(D // LANES):
                    v = _pb[r, pl.ds(c * LANES, LANES)]
                    for b in range(B):
                        plsc.addupdate(
                            _tb.at[b * CH + r, pl.ds(c * LANES, LANES)], v)
                return carry

            lax.fori_loop(0, CH, row_body, 0)

            hs = []
            for b in range(B):
                h = pltpu.make_async_copy(
                    tb.at[pl.ds(b * CH, CH)],
                    out_hbm.at[pl.ds(b * S + p0 + q * CH, CH), :],
                    ssems[g])
                h.start()
                hs.append(h)
            pend_store[g] = hs

            if q + NGRP < n_q:
                issue_chunk(q + NGRP)

        for g in range(NGRP):
            if pend_store[g] is not None:
                for h in pend_store[g]:
                    h.wait()

    return emb_kernel, n_q, rows_g, nw


_BUILT = None


def kernel(input_ids, token_table, pos_table):
    global _BUILT
    if _BUILT is None:
        _BUILT = _build_kernel()
    emb_kernel, n_q, rows_g, nw = _BUILT
    ids = input_ids.astype(jnp.int32).reshape(B, nw, n_q, CH)
    ids = ids.transpose(1, 2, 0, 3).reshape(nw * n_q, rows_g)
    out = emb_kernel(ids, token_table, pos_table)
    return out.reshape(B, S, D)
